# Initial kernel scaffold; baseline (speedup 1.0000x reference)
import jax
import jax.numpy as jnp
from jax.experimental import pallas as pl


def kernel(uid, trust_segment_ids, trusteeid, user_idx, item_pos_idx, item_neg_idx, userEmbed, itemEmbed, key_param, mem_param, w1_w, w1_b, w3_w, w3_b, b, h_w):
    raise NotImplementedError("write your pallas kernel here")


# speedup vs baseline: 5.6721x; 5.6721x over previous
import functools

import jax
import jax.numpy as jnp
from jax import lax
from jax.experimental import pallas as pl
from jax.experimental.pallas import tpu as pltpu
from jax.experimental.pallas import tpu_sc as plsc

NC, NS = 2, 16
NW = NC * NS
L = 16
H = 64
W = 80
CH = 128


def _sc_mesh():
    return plsc.VectorSubcoreMesh(core_axis_name="c", subcore_axis_name="s",
                                  num_cores=NC, num_subcores=NS)


_SC_PARAMS = pltpu.CompilerParams(needs_layout_passes=False, use_tc_tiling_on_sc=False)


@functools.lru_cache(maxsize=None)
def _make_gather1(B, T, I):
    EPW = T // NW
    IPW = I // NW
    UPW = B // NW
    n_ec = EPW // CH
    n_ic = IPW // CH

    @functools.partial(
        pl.kernel, mesh=_sc_mesh(),
        compiler_params=_SC_PARAMS,
        out_type=(
            jax.ShapeDtypeStruct((B, H), jnp.float32),
            jax.ShapeDtypeStruct((T, H), jnp.float32),
            jax.ShapeDtypeStruct((T, H), jnp.float32),
            jax.ShapeDtypeStruct((I, H), jnp.float32),
            jax.ShapeDtypeStruct((I, H), jnp.float32),
        ),
        scratch_types=[
            pltpu.VMEM((B,), jnp.int32),
            pltpu.VMEM((EPW,), jnp.int32),
            pltpu.VMEM((EPW,), jnp.int32),
            pltpu.VMEM((IPW,), jnp.int32),
            pltpu.VMEM((IPW,), jnp.int32),
            pltpu.VMEM((CH,), jnp.int32),
            pltpu.VMEM((CH,), jnp.int32),
            pltpu.VMEM((CH, H), jnp.float32),
            pltpu.VMEM((CH, H), jnp.float32),
            pltpu.VMEM((CH, H), jnp.float32),
            pltpu.VMEM((CH, H), jnp.float32),
            pltpu.SemaphoreType.DMA,
            pltpu.SemaphoreType.DMA,
            pltpu.SemaphoreType.DMA,
            pltpu.SemaphoreType.DMA,
            pltpu.SemaphoreType.DMA,
            pltpu.SemaphoreType.DMA,
            pltpu.SemaphoreType.DMA,
            pltpu.SemaphoreType.DMA,
        ],
    )
    def k(uid, seg, teid, ipid, inid, ue, ie, U, US, TE, IP, IN,
          uid_v, seg_v, teid_v, ipid_v, inid_v, tid0, tid1,
          ra0, ra1, rb0, rb1, ga0, ga1, gb0, gb1, wa0, wa1, wb0, wb1):
        tid = (tid0, tid1)
        rA = (ra0, ra1)
        rB = (rb0, rb1)
        gA = (ga0, ga1)
        gB = (gb0, gb1)
        wA = (wa0, wa1)
        wB = (wb0, wb1)
        wid = lax.axis_index("s") * NC + lax.axis_index("c")
        eb = wid * EPW
        ib = wid * IPW
        ub = wid * UPW
        pltpu.sync_copy(uid, uid_v)
        pltpu.sync_copy(seg.at[pl.ds(eb, EPW)], seg_v)
        pltpu.sync_copy(teid.at[pl.ds(eb, EPW)], teid_v)
        pltpu.sync_copy(ipid.at[pl.ds(ib, IPW)], ipid_v)
        pltpu.sync_copy(inid.at[pl.ds(ib, IPW)], inid_v)
        pltpu.async_copy(ue.at[uid_v.at[pl.ds(ub, UPW)]], ra0, ga0).wait()
        pltpu.sync_copy(ra0, U.at[pl.ds(ub, UPW)])

        def fire_e(b, c):
            for j in range(CH // L):
                v = seg_v[pl.ds(c * CH + j * L, L)]
                tid[b][pl.ds(j * L, L)] = plsc.load_gather(uid_v, [v])
            pltpu.async_copy(ue.at[tid[b]], rA[b], gA[b])
            pltpu.async_copy(ue.at[teid_v.at[pl.ds(c * CH, CH)]], rB[b], gB[b])

        fire_e(0, 0)

        def epair(p, carry):
            for b in range(2):
                c = 2 * p + b
                nb = 1 - b

                @pl.when(c + 1 < n_ec)
                def _():
                    @pl.when(c >= 1)
                    def _():
                        pltpu.make_async_copy(rA[nb], US.at[pl.ds(eb, CH)],
                                              wA[nb]).wait()
                        pltpu.make_async_copy(rB[nb], TE.at[pl.ds(eb, CH)],
                                              wB[nb]).wait()
                    fire_e(nb, c + 1)

                pltpu.make_async_copy(ue.at[tid[b]], rA[b], gA[b]).wait()
                pltpu.async_copy(rA[b], US.at[pl.ds(eb + c * CH, CH)], wA[b])
                pltpu.make_async_copy(ue.at[teid_v.at[pl.ds(c * CH, CH)]],
                                      rB[b], gB[b]).wait()
                pltpu.async_copy(rB[b], TE.at[pl.ds(eb + c * CH, CH)], wB[b])
            return carry
        lax.fori_loop(0, n_ec // 2, epair, 0)
        for b in range(2):
            pltpu.make_async_copy(rA[b], US.at[pl.ds(eb, CH)], wA[b]).wait()
            pltpu.make_async_copy(rB[b], TE.at[pl.ds(eb, CH)], wB[b]).wait()

        def fire_i(b, c):
            pltpu.async_copy(ie.at[ipid_v.at[pl.ds(c * CH, CH)]], rA[b], gA[b])
            pltpu.async_copy(ie.at[inid_v.at[pl.ds(c * CH, CH)]], rB[b], gB[b])

        fire_i(0, 0)

        def ipair(p, carry):
            for b in range(2):
                c = 2 * p + b
                nb = 1 - b

                @pl.when(c + 1 < n_ic)
                def _():
                    @pl.when(c >= 1)
                    def _():
                        pltpu.make_async_copy(rA[nb], IP.at[pl.ds(ib, CH)],
                                              wA[nb]).wait()
                        pltpu.make_async_copy(rB[nb], IN.at[pl.ds(ib, CH)],
                                              wB[nb]).wait()
                    fire_i(nb, c + 1)

                pltpu.make_async_copy(ie.at[ipid_v.at[pl.ds(c * CH, CH)]],
                                      rA[b], gA[b]).wait()
                pltpu.async_copy(rA[b], IP.at[pl.ds(ib + c * CH, CH)], wA[b])
                pltpu.make_async_copy(ie.at[inid_v.at[pl.ds(c * CH, CH)]],
                                      rB[b], gB[b]).wait()
                pltpu.async_copy(rB[b], IN.at[pl.ds(ib + c * CH, CH)], wB[b])
            return carry
        lax.fori_loop(0, n_ic // 2, ipair, 0)
        for b in range(2):
            pltpu.make_async_copy(rA[b], IP.at[pl.ds(ib, CH)], wA[b]).wait()
            pltpu.make_async_copy(rB[b], IN.at[pl.ds(ib, CH)], wB[b]).wait()

    return k


def _edge_body(te_ref, us_ref, kp_ref, mp_ref, w1t_ref, w3t_ref, bias_ref,
               hwt_ref, out_ref):
    te = te_ref[...]
    us = us_ref[...]
    nt = jnp.sqrt(jnp.sum(te * te, axis=1, keepdims=True))
    nu = jnp.sqrt(jnp.sum(us * us, axis=1, keepdims=True))
    s = us * te / (nu * nt)
    z = jnp.dot(s, kp_ref[...], preferred_element_type=jnp.float32)
    z = z - jnp.max(z, axis=1, keepdims=True)
    ez = jnp.exp(z)
    a = ez / jnp.sum(ez, axis=1, keepdims=True)
    g = jnp.dot(a, mp_ref[...], preferred_element_type=jnp.float32)
    f = te * g
    pre = (jnp.dot(us, w1t_ref[...], preferred_element_type=jnp.float32)
           + jnp.dot(f, w3t_ref[...], preferred_element_type=jnp.float32)
           + bias_ref[...])
    beta = jnp.dot(jnp.maximum(pre, 0.0), hwt_ref[...],
                   preferred_element_type=jnp.float32)
    e = jnp.exp(beta)
    out_ref[:, 0:H] = f * e
    lane = lax.broadcasted_iota(jnp.int32, (te.shape[0], W - H), 1)
    out_ref[:, H:W] = jnp.where(lane == 0, e, 0.0)


def _edge_call(T, interpret=False):
    RB = 2048
    return pl.pallas_call(
        _edge_body,
        grid=(T // RB,),
        in_specs=[
            pl.BlockSpec((RB, H), lambda i: (i, 0)),
            pl.BlockSpec((RB, H), lambda i: (i, 0)),
            pl.BlockSpec((H, 8), lambda i: (0, 0)),
            pl.BlockSpec((8, H), lambda i: (0, 0)),
            pl.BlockSpec((H, 32), lambda i: (0, 0)),
            pl.BlockSpec((H, 32), lambda i: (0, 0)),
            pl.BlockSpec((1, 32), lambda i: (0, 0)),
            pl.BlockSpec((32, 1), lambda i: (0, 0)),
        ],
        out_specs=pl.BlockSpec((RB, W), lambda i: (i, 0)),
        out_shape=jax.ShapeDtypeStruct((T, W), jnp.float32),
        interpret=interpret,
    )


@functools.lru_cache(maxsize=None)
def _make_scatter(B, T):
    EPW = T // NW
    n_ec = EPW // CH
    RPT = B // NS

    @functools.partial(
        pl.kernel, mesh=_sc_mesh(),
        compiler_params=_SC_PARAMS,
        out_type=jax.ShapeDtypeStruct((NC, B, W), jnp.float32),
        scratch_types=[
            pltpu.VMEM((CH, W), jnp.float32),
            pltpu.VMEM((CH, W), jnp.float32),
            pltpu.VMEM((n_ec, CH), jnp.int32),
            pltpu.VMEM((RPT, W), jnp.float32),
            pltpu.VMEM_SHARED((B, W), jnp.float32),
            pltpu.SemaphoreType.DMA,
            pltpu.SemaphoreType.DMA,
        ],
    )
    def k(fee, seg2d, zrows, out, fee0, fee1, seg_v, stage_v, acc, ls0, ls1):
        fb = (fee0, fee1)
        ls = (ls0, ls1)
        cid = lax.axis_index("c")
        sid = lax.axis_index("s")
        wid = sid * NC + cid
        pltpu.sync_copy(zrows, stage_v)
        pltpu.sync_copy(stage_v, acc.at[pl.ds(sid * RPT, RPT)])
        plsc.subcore_barrier()
        pltpu.sync_copy(seg2d.at[pl.ds(wid * n_ec, n_ec)], seg_v)

        pltpu.async_copy(fee.at[pl.ds(wid * EPW, CH)], fee0, ls0)

        def pair(p, carry):
            for b in range(2):
                c = 2 * p + b
                nb = 1 - b

                @pl.when(c + 1 < n_ec)
                def _():
                    pltpu.async_copy(
                        fee.at[pl.ds(wid * EPW + (c + 1) * CH, CH)],
                        fb[nb], ls[nb])

                pltpu.make_async_copy(fee.at[pl.ds(wid * EPW, CH)],
                                      fb[b], ls[b]).wait()
                pltpu.sync_copy(fb[b], acc.at[seg_v.at[c]], add=True)
            return carry
        lax.fori_loop(0, n_ec // 2, pair, 0)
        plsc.subcore_barrier()
        pltpu.sync_copy(acc.at[pl.ds(sid * RPT, RPT)], stage_v)
        pltpu.sync_copy(stage_v, out.at[cid, pl.ds(sid * RPT, RPT)])

    return k


def _finalize_body(p2_ref, u_ref, v_ref):
    facc = p2_ref[0, :, 0:H] + p2_ref[1, :, 0:H]
    w = p2_ref[0, :, H:H + 1] + p2_ref[1, :, H:H + 1]
    v_ref[...] = jnp.where(w > 0, facc / jnp.where(w > 0, w, 1.0), 0.0) \
        + u_ref[...]


def _finalize_call(B, interpret=False):
    return pl.pallas_call(
        _finalize_body,
        out_shape=jax.ShapeDtypeStruct((B, H), jnp.float32),
        interpret=interpret,
    )


@functools.lru_cache(maxsize=None)
def _make_gather2(B, I):
    IPW = I // NW
    n_ic = IPW // CH

    @functools.partial(
        pl.kernel, mesh=_sc_mesh(),
        compiler_params=_SC_PARAMS,
        out_type=jax.ShapeDtypeStruct((I, H), jnp.float32),
        scratch_types=[
            pltpu.VMEM((IPW,), jnp.int32),
            pltpu.VMEM((CH, H), jnp.float32),
            pltpu.VMEM((CH, H), jnp.float32),
            pltpu.SemaphoreType.DMA,
            pltpu.SemaphoreType.DMA,
            pltpu.SemaphoreType.DMA,
            pltpu.SemaphoreType.DMA,
        ],
    )
    def k(uidx, v_tab, VE, uidx_v, r0, r1, g0, g1, w0, w1):
        rr = (r0, r1)
        gs = (g0, g1)
        ws = (w0, w1)
        wid = lax.axis_index("s") * NC + lax.axis_index("c")
        ib = wid * IPW
        pltpu.sync_copy(uidx.at[pl.ds(ib, IPW)], uidx_v)

        pltpu.async_copy(v_tab.at[uidx_v.at[pl.ds(0, CH)]], r0, g0)

        def pair(p, carry):
            for b in range(2):
                c = 2 * p + b
                nb = 1 - b

                @pl.when(c + 1 < n_ic)
                def _():
                    @pl.when(c >= 1)
                    def _():
                        pltpu.make_async_copy(rr[nb], VE.at[pl.ds(ib, CH)],
                                              ws[nb]).wait()
                    pltpu.async_copy(
                        v_tab.at[uidx_v.at[pl.ds((c + 1) * CH, CH)]],
                        rr[nb], gs[nb])

                pltpu.make_async_copy(
                    v_tab.at[uidx_v.at[pl.ds(c * CH, CH)]], rr[b],
                    gs[b]).wait()
                pltpu.async_copy(rr[b], VE.at[pl.ds(ib + c * CH, CH)], ws[b])
            return carry
        lax.fori_loop(0, n_ic // 2, pair, 0)
        for b in range(2):
            pltpu.make_async_copy(rr[b], VE.at[pl.ds(ib, CH)], ws[b]).wait()

    return k


def _dot_body(ve_ref, ip_ref, in_ref, op_ref, on_ref):
    v = ve_ref[...]
    op_ref[0, 0, :] = jnp.sum(v * ip_ref[...], axis=1)
    on_ref[0, 0, :] = jnp.sum(v * in_ref[...], axis=1)


def _dot_call(I, interpret=False):
    RB = min(8192, I)
    nb = I // RB
    return pl.pallas_call(
        _dot_body,
        grid=(nb,),
        in_specs=[
            pl.BlockSpec((RB, H), lambda i: (i, 0)),
            pl.BlockSpec((RB, H), lambda i: (i, 0)),
            pl.BlockSpec((RB, H), lambda i: (i, 0)),
        ],
        out_specs=[
            pl.BlockSpec((1, 1, RB), lambda i: (i, 0, 0)),
            pl.BlockSpec((1, 1, RB), lambda i: (i, 0, 0)),
        ],
        out_shape=[
            jax.ShapeDtypeStruct((nb, 1, RB), jnp.float32),
            jax.ShapeDtypeStruct((nb, 1, RB), jnp.float32),
        ],
        interpret=interpret,
    )


def kernel(uid, trust_segment_ids, trusteeid, user_idx, item_pos_idx,
           item_neg_idx, userEmbed, itemEmbed, key_param, mem_param,
           w1_w, w1_b, w3_w, w3_b, b, h_w):
    B = uid.shape[0]
    T = trust_segment_ids.shape[0]
    I = user_idx.shape[0]

    uid = uid.astype(jnp.int32)
    seg = trust_segment_ids.astype(jnp.int32)
    teid = trusteeid.astype(jnp.int32)
    uidx = user_idx.astype(jnp.int32)
    ipid = item_pos_idx.astype(jnp.int32)
    inid = item_neg_idx.astype(jnp.int32)

    U, US, TE, IP, IN = _make_gather1(B, T, I)(
        uid, seg, teid, ipid, inid, userEmbed, itemEmbed)

    bias = (w1_b + w3_b + b.reshape(-1)).reshape(1, -1)
    FEE = _edge_call(T)(TE, US, key_param, mem_param, w1_w.T, w3_w.T,
                        bias, h_w.T)

    zrows = jnp.zeros((B // NS, W), jnp.float32)
    P2 = _make_scatter(B, T)(FEE, seg.reshape(-1, CH), zrows)

    V = _finalize_call(B)(P2, U)
    VE = _make_gather2(B, I)(uidx, V)
    pp, pn = _dot_call(I)(VE, IP, IN)
    return pp.reshape(-1), pn.reshape(-1)
